# baseline (speedup 1.0000x reference)
import functools

import jax
import jax.numpy as jnp
from jax import lax
from jax.experimental import pallas as pl
from jax.experimental.pallas import tpu as pltpu
from jax.experimental.pallas import tpu_sc as plsc

_B, _N, _K, _D = 2, 2048, 32, 1024
_NW = 32
_RPW = _N // _NW
_RBLK = 32


def _mm_body(a_ref, b_ref, s_ref):
    s_ref[...] = lax.dot_general(
        a_ref[...].astype(jnp.bfloat16), b_ref[...].astype(jnp.bfloat16),
        dimension_numbers=(((1,), (1,)), ((), ())),
        preferred_element_type=jnp.float32,
    )


def _dense_scores(A1, B1):
    bm = 256
    return pl.pallas_call(
        _mm_body,
        grid=(_N // bm,),
        in_specs=[
            pl.BlockSpec((bm, _D), lambda m: (m, 0)),
            pl.BlockSpec((_N, _D), lambda m: (0, 0)),
        ],
        out_specs=pl.BlockSpec((bm, _N), lambda m: (m, 0)),
        out_shape=jax.ShapeDtypeStruct((_N, _N), jnp.float32),
    )(A1, B1)


def _sc_sample_body(s_hbm, idx_hbm, out_hbm, s_v, idx_v, out_v):
    wid = lax.axis_index("s") * 2 + lax.axis_index("c")
    row0 = wid * _RPW

    def do_block(blk, _):
        base = row0 + blk * _RBLK
        pltpu.sync_copy(s_hbm.at[pl.ds(base, _RBLK)], s_v)
        pltpu.sync_copy(idx_hbm.at[pl.ds(base, _RBLK)], idx_v)

        def do_row(r, _):
            rvec = jnp.broadcast_to(r, (16,)).astype(jnp.int32)
            for h in range(_K // 16):
                iv = idx_v[r, pl.ds(h * 16, 16)]
                out_v[r, pl.ds(h * 16, 16)] = plsc.load_gather(s_v, [rvec, iv])
            return 0

        lax.fori_loop(0, _RBLK, do_row, 0)
        pltpu.sync_copy(out_v, out_hbm.at[pl.ds(base, _RBLK)])
        return 0

    lax.fori_loop(0, _RPW // _RBLK, do_block, 0)


def _sc_sample(S2, idx2):
    mesh = plsc.VectorSubcoreMesh(core_axis_name="c", subcore_axis_name="s")
    return pl.kernel(
        _sc_sample_body,
        out_type=jax.ShapeDtypeStruct((_N, _K), jnp.float32),
        mesh=mesh,
        scratch_types=[
            pltpu.VMEM((_RBLK, _N), jnp.float32),
            pltpu.VMEM((_RBLK, _K), jnp.int32),
            pltpu.VMEM((_RBLK, _K), jnp.float32),
        ],
        compiler_params=pltpu.CompilerParams(
            use_tc_tiling_on_sc=True, needs_layout_passes=False),
    )(S2, idx2)


def kernel(A, B, index):
    outs = []
    for b in range(_B):
        S = _dense_scores(A[b], B[b])
        outs.append(_sc_sample(S, index[b]))
    return jnp.stack(outs)


# speedup vs baseline: 1.3104x; 1.3104x over previous
import functools

import jax
import jax.numpy as jnp
from jax import lax
from jax.experimental import pallas as pl
from jax.experimental.pallas import tpu as pltpu
from jax.experimental.pallas import tpu_sc as plsc

_B, _N, _K, _D = 2, 2048, 32, 1024
_ROWS = _B * _N
_NW = 32
_BM = 256
_WBLK = 16


def _mm_body(a_ref, b_ref, w_ref):
    s = lax.dot_general(
        a_ref[0].astype(jnp.bfloat16), b_ref[0].astype(jnp.bfloat16),
        dimension_numbers=(((1,), (1,)), ((), ())),
        preferred_element_type=jnp.float32,
    )
    u = lax.bitcast_convert_type(s.astype(jnp.bfloat16), jnp.uint16)
    lo = u[: _BM // 2].astype(jnp.uint32)
    hi = u[_BM // 2:].astype(jnp.uint32)
    w_ref[...] = lax.bitcast_convert_type(lo | (hi << 16), jnp.int32)[None]


def _dense_scores_packed(A, B):
    return pl.pallas_call(
        _mm_body,
        grid=(_B, _N // _BM),
        in_specs=[
            pl.BlockSpec((1, _BM, _D), lambda b, m: (b, m, 0)),
            pl.BlockSpec((1, _N, _D), lambda b, m: (b, 0, 0)),
        ],
        out_specs=pl.BlockSpec((1, _BM // 2, _N), lambda b, m: (b, m, 0)),
        out_shape=jax.ShapeDtypeStruct((_B, _N // 2, _N), jnp.int32),
    )(A, B)


def _sc_sample_body(w_hbm, idx_hbm, out_hbm, w_v, idx_v, out_v):
    wid = lax.axis_index("s") * 2 + lax.axis_index("c")
    g = wid // 2
    half = wid % 2

    def do_block(blk, _):
        wb = g * 128 + half * 64 + blk * _WBLK
        r_lo = g * 256 + half * 64 + blk * _WBLK
        r_hi = r_lo + 128
        pltpu.sync_copy(w_hbm.at[pl.ds(wb, _WBLK)], w_v)
        pltpu.sync_copy(idx_hbm.at[pl.ds(r_lo, _WBLK)],
                        idx_v.at[pl.ds(0, _WBLK)])
        pltpu.sync_copy(idx_hbm.at[pl.ds(r_hi, _WBLK)],
                        idx_v.at[pl.ds(_WBLK, _WBLK)])

        def do_word(t, _):
            wvec = jnp.broadcast_to(t, (16,)).astype(jnp.int32)
            for h in range(_K // 16):
                iv = idx_v[t, pl.ds(h * 16, 16)]
                w = plsc.load_gather(w_v, [wvec, iv])
                out_v[t, pl.ds(h * 16, 16)] = lax.bitcast_convert_type(
                    w << 16, jnp.float32)
                iv2 = idx_v[t + _WBLK, pl.ds(h * 16, 16)]
                w2 = plsc.load_gather(w_v, [wvec, iv2])
                out_v[t + _WBLK, pl.ds(h * 16, 16)] = lax.bitcast_convert_type(
                    w2 & jnp.int32(-65536), jnp.float32)
            return 0

        lax.fori_loop(0, _WBLK, do_word, 0)
        pltpu.sync_copy(out_v.at[pl.ds(0, _WBLK)],
                        out_hbm.at[pl.ds(r_lo, _WBLK)])
        pltpu.sync_copy(out_v.at[pl.ds(_WBLK, _WBLK)],
                        out_hbm.at[pl.ds(r_hi, _WBLK)])
        return 0

    lax.fori_loop(0, 64 // _WBLK, do_block, 0)


def _sc_sample(W2, idx2):
    mesh = plsc.VectorSubcoreMesh(core_axis_name="c", subcore_axis_name="s")
    return pl.kernel(
        _sc_sample_body,
        out_type=jax.ShapeDtypeStruct((_ROWS, _K), jnp.float32),
        mesh=mesh,
        scratch_types=[
            pltpu.VMEM((_WBLK, _N), jnp.int32),
            pltpu.VMEM((2 * _WBLK, _K), jnp.int32),
            pltpu.VMEM((2 * _WBLK, _K), jnp.float32),
        ],
        compiler_params=pltpu.CompilerParams(
            use_tc_tiling_on_sc=True, needs_layout_passes=False),
    )(W2, idx2)


def kernel(A, B, index):
    W = _dense_scores_packed(A, B)
    W2 = W.reshape(_ROWS // 2, _N)
    idx2 = index.reshape(_ROWS, _K)
    C2 = _sc_sample(W2, idx2)
    return C2.reshape(_B, _N, _K)


# speedup vs baseline: 1.4161x; 1.0807x over previous
import functools

import jax
import jax.numpy as jnp
from jax import lax
from jax.experimental import pallas as pl
from jax.experimental.pallas import tpu as pltpu
from jax.experimental.pallas import tpu_sc as plsc

_B, _N, _K, _D = 2, 2048, 32, 1024
_ROWS = _B * _N
_NW = 32
_BM = 256
_WBLK = 16


def _mm_body(a_ref, b_ref, w_ref):
    s = lax.dot_general(
        a_ref[0].astype(jnp.bfloat16), b_ref[0].astype(jnp.bfloat16),
        dimension_numbers=(((1,), (1,)), ((), ())),
        preferred_element_type=jnp.float32,
    )
    u = lax.bitcast_convert_type(s.astype(jnp.bfloat16), jnp.uint16)
    lo = u[: _BM // 2].astype(jnp.uint32)
    hi = u[_BM // 2:].astype(jnp.uint32)
    w_ref[...] = lax.bitcast_convert_type(lo | (hi << 16), jnp.int32)[None]


def _dense_scores_packed(A, B):
    return pl.pallas_call(
        _mm_body,
        grid=(_B, _N // _BM),
        in_specs=[
            pl.BlockSpec((1, _BM, _D), lambda b, m: (b, m, 0)),
            pl.BlockSpec((1, _N, _D), lambda b, m: (b, 0, 0)),
        ],
        out_specs=pl.BlockSpec((1, _BM // 2, _N), lambda b, m: (b, m, 0)),
        out_shape=jax.ShapeDtypeStruct((_B, _N // 2, _N), jnp.int32),
    )(A, B)


_NBLK = 64 // _WBLK


def _sc_sample_body(w_hbm, idx_hbm, out_hbm, w_v, idx_v, out_v, sem):
    wid = lax.axis_index("s") * 2 + lax.axis_index("c")
    g = wid // 2
    half = wid % 2

    def w_copy(blk):
        wb = g * 128 + half * 64 + blk * _WBLK
        return pltpu.make_async_copy(
            w_hbm.at[pl.ds(wb, _WBLK)], w_v.at[blk % 2], sem)

    first = w_copy(0)
    first.start()
    copies = [first]
    for blk in range(_NBLK):
        if blk + 1 < _NBLK:
            nxt = w_copy(blk + 1)
            nxt.start()
            copies.append(nxt)
        r_lo = g * 256 + half * 64 + blk * _WBLK
        r_hi = r_lo + 128
        b_lo, i_lo = r_lo // _N, r_lo % _N
        b_hi, i_hi = r_hi // _N, r_hi % _N
        pltpu.sync_copy(idx_hbm.at[b_lo, pl.ds(i_lo, _WBLK)],
                        idx_v.at[pl.ds(0, _WBLK)])
        pltpu.sync_copy(idx_hbm.at[b_hi, pl.ds(i_hi, _WBLK)],
                        idx_v.at[pl.ds(_WBLK, _WBLK)])
        copies[blk].wait()

        def do_word(t, _):
            wvec = jnp.broadcast_to(t, (16,)).astype(jnp.int32)
            for h in range(_K // 16):
                iv = idx_v[t, pl.ds(h * 16, 16)]
                w = plsc.load_gather(w_v.at[blk % 2], [wvec, iv])
                out_v[t, pl.ds(h * 16, 16)] = lax.bitcast_convert_type(
                    w << 16, jnp.float32)
                iv2 = idx_v[t + _WBLK, pl.ds(h * 16, 16)]
                w2 = plsc.load_gather(w_v.at[blk % 2], [wvec, iv2])
                out_v[t + _WBLK, pl.ds(h * 16, 16)] = lax.bitcast_convert_type(
                    w2 & jnp.int32(-65536), jnp.float32)
            return 0

        lax.fori_loop(0, _WBLK, do_word, 0)
        pltpu.sync_copy(out_v.at[pl.ds(0, _WBLK)],
                        out_hbm.at[b_lo, pl.ds(i_lo, _WBLK)])
        pltpu.sync_copy(out_v.at[pl.ds(_WBLK, _WBLK)],
                        out_hbm.at[b_hi, pl.ds(i_hi, _WBLK)])


def _sc_sample(W2, index):
    mesh = plsc.VectorSubcoreMesh(core_axis_name="c", subcore_axis_name="s")
    return pl.kernel(
        _sc_sample_body,
        out_type=jax.ShapeDtypeStruct((_B, _N, _K), jnp.float32),
        mesh=mesh,
        scratch_types=[
            pltpu.VMEM((2, _WBLK, _N), jnp.int32),
            pltpu.VMEM((2 * _WBLK, _K), jnp.int32),
            pltpu.VMEM((2 * _WBLK, _K), jnp.float32),
            pltpu.SemaphoreType.DMA,
        ],
        compiler_params=pltpu.CompilerParams(
            use_tc_tiling_on_sc=True, needs_layout_passes=False),
    )(W2, index)


def kernel(A, B, index):
    W = _dense_scores_packed(A, B)
    W2 = W.reshape(_ROWS // 2, _N)
    return _sc_sample(W2, index)
